# Initial kernel scaffold; baseline (speedup 1.0000x reference)
import functools

import jax
import jax.numpy as jnp
from jax import lax
from jax.experimental import pallas as pl
from jax.experimental.pallas import tpu as pltpu
from jax.experimental.pallas import tpu_sc as plsc

VOCAB = 100000
D = 64
B = 4096
L = 200
HID = 100

_info = plsc.get_sparse_core_info()
NC, NS, LANES = _info.num_cores, _info.num_subcores, _info.num_lanes
NW = NC * NS
BPW = B // NW


def _sc_pool(x_flat, table):
    mesh = plsc.VectorSubcoreMesh(core_axis_name="c", subcore_axis_name="s")

    @functools.partial(
        pl.kernel,
        out_type=jax.ShapeDtypeStruct((B, D), jnp.float32),
        mesh=mesh,
        scratch_types=[
            pltpu.VMEM((BPW * L,), jnp.int32),
            pltpu.VMEM((L, D), jnp.float32),
            pltpu.VMEM((BPW, D), jnp.float32),
            pltpu.SemaphoreType.DMA,
        ],
    )
    def k(x_hbm, table_hbm, out_hbm, idx_v, rows_v, pooled_v, sem):
        wid = lax.axis_index("s") * NC + lax.axis_index("c")
        base = wid * (BPW * L)
        pltpu.sync_copy(x_hbm.at[pl.ds(base, BPW * L)], idx_v)

        def row_body(j, _):
            off = pl.multiple_of(j * L, 8)
            c1 = pltpu.async_copy(
                table_hbm.at[idx_v.at[pl.ds(off, 96)]],
                rows_v.at[pl.ds(0, 96)], sem)
            c2 = pltpu.async_copy(
                table_hbm.at[idx_v.at[pl.ds(off + 96, 104)]],
                rows_v.at[pl.ds(96, 104)], sem)
            c1.wait()
            c2.wait()

            def acc_body(l, carry):
                a0, a1, a2, a3 = carry
                a0 = a0 + rows_v[l, pl.ds(0, 16)]
                a1 = a1 + rows_v[l, pl.ds(16, 16)]
                a2 = a2 + rows_v[l, pl.ds(32, 16)]
                a3 = a3 + rows_v[l, pl.ds(48, 16)]
                return a0, a1, a2, a3

            z = jnp.zeros((16,), jnp.float32)
            a0, a1, a2, a3 = lax.fori_loop(0, L, acc_body, (z, z, z, z))
            scale = jnp.float32(1.0 / L)
            pooled_v[j, pl.ds(0, 16)] = a0 * scale
            pooled_v[j, pl.ds(16, 16)] = a1 * scale
            pooled_v[j, pl.ds(32, 16)] = a2 * scale
            pooled_v[j, pl.ds(48, 16)] = a3 * scale
            return 0

        lax.fori_loop(0, BPW, row_body, 0)
        pltpu.sync_copy(pooled_v, out_hbm.at[pl.ds(wid * BPW, BPW)])

    return k(x_flat, table)


def _mlp_body(p_ref, w1_ref, b1_ref, w2_ref, b2_ref, o_ref):
    h = jnp.dot(p_ref[...], w1_ref[...], preferred_element_type=jnp.float32)
    h = jnp.maximum(h + b1_ref[...], 0.0)
    o = jnp.dot(h, w2_ref[...], preferred_element_type=jnp.float32)
    o_ref[...] = jax.nn.sigmoid(o + b2_ref[...])


def _tc_mlp(pooled, W1, b1, W2, b2):
    return pl.pallas_call(
        _mlp_body,
        out_shape=jax.ShapeDtypeStruct((B, 1), jnp.float32),
    )(pooled, W1, b1.reshape(1, HID), W2, b2.reshape(1, 1))


def kernel(x, table, W1, b1, W2, b2):
    pooled = _sc_pool(x.reshape(-1), table)
    return _tc_mlp(pooled, W1, b1, W2, b2)


# speedup vs baseline: 8.7071x; 8.7071x over previous
import functools

import jax
import jax.numpy as jnp
from jax import lax
from jax.experimental import pallas as pl
from jax.experimental.pallas import tpu as pltpu
from jax.experimental.pallas import tpu_sc as plsc

VOCAB = 100000
D = 64
B = 4096
L = 200
HID = 100

_info = plsc.get_sparse_core_info()
NC, NS, LANES = _info.num_cores, _info.num_subcores, _info.num_lanes
NW = NC * NS
BPW = B // NW


def _sc_pool(x_flat, table):
    mesh = plsc.VectorSubcoreMesh(core_axis_name="c", subcore_axis_name="s")

    @functools.partial(
        pl.kernel,
        out_type=jax.ShapeDtypeStruct((B, D), jnp.float32),
        mesh=mesh,
        scratch_types=[
            pltpu.VMEM((BPW * L,), jnp.int32),
            pltpu.VMEM((L, D), jnp.float32),
            pltpu.VMEM((BPW, D), jnp.float32),
            pltpu.SemaphoreType.DMA,
        ],
        compiler_params=pltpu.CompilerParams(use_tc_tiling_on_sc=False),
    )
    def k(x_hbm, table_hbm, out_hbm, idx_v, rows_v, pooled_v, sem):
        wid = lax.axis_index("s") * NC + lax.axis_index("c")
        base = wid * (BPW * L)
        pltpu.sync_copy(x_hbm.at[pl.ds(base, BPW * L)], idx_v)

        def row_body(j, _):
            off = pl.multiple_of(j * L, 8)
            c1 = pltpu.async_copy(
                table_hbm.at[idx_v.at[pl.ds(off, 96)]],
                rows_v.at[pl.ds(0, 96)], sem)
            c2 = pltpu.async_copy(
                table_hbm.at[idx_v.at[pl.ds(off + 96, 104)]],
                rows_v.at[pl.ds(96, 104)], sem)
            c1.wait()
            c2.wait()

            def acc_body(l, carry):
                a0, a1, a2, a3 = carry
                a0 = a0 + rows_v[l, pl.ds(0, 16)]
                a1 = a1 + rows_v[l, pl.ds(16, 16)]
                a2 = a2 + rows_v[l, pl.ds(32, 16)]
                a3 = a3 + rows_v[l, pl.ds(48, 16)]
                return a0, a1, a2, a3

            z = jnp.zeros((16,), jnp.float32)
            a0, a1, a2, a3 = lax.fori_loop(0, L, acc_body, (z, z, z, z))
            scale = jnp.float32(1.0 / L)
            pooled_v[j, pl.ds(0, 16)] = a0 * scale
            pooled_v[j, pl.ds(16, 16)] = a1 * scale
            pooled_v[j, pl.ds(32, 16)] = a2 * scale
            pooled_v[j, pl.ds(48, 16)] = a3 * scale
            return 0

        lax.fori_loop(0, BPW, row_body, 0)
        pltpu.sync_copy(pooled_v, out_hbm.at[pl.ds(wid * BPW, BPW)])

    return k(x_flat, table)


def _mlp_body(p_ref, w1_ref, b1_ref, w2_ref, b2_ref, o_ref):
    h = jnp.dot(p_ref[...], w1_ref[...], preferred_element_type=jnp.float32)
    h = jnp.maximum(h + b1_ref[...], 0.0)
    o = jnp.dot(h, w2_ref[...], preferred_element_type=jnp.float32)
    o_ref[...] = jax.nn.sigmoid(o + b2_ref[...])


def _tc_mlp(pooled, W1, b1, W2, b2):
    return pl.pallas_call(
        _mlp_body,
        out_shape=jax.ShapeDtypeStruct((B, 1), jnp.float32),
    )(pooled, W1, b1.reshape(1, HID), W2, b2.reshape(1, 1))


def kernel(x, table, W1, b1, W2, b2):
    pooled = _sc_pool(x.reshape(-1), table)
    return _tc_mlp(pooled, W1, b1, W2, b2)


# speedup vs baseline: 14.1309x; 1.6229x over previous
import functools

import jax
import jax.numpy as jnp
from jax import lax
from jax.experimental import pallas as pl
from jax.experimental.pallas import tpu as pltpu
from jax.experimental.pallas import tpu_sc as plsc

VOCAB = 100000
D = 64
B = 4096
L = 200
HID = 100

_info = plsc.get_sparse_core_info()
NC, NS, LANES = _info.num_cores, _info.num_subcores, _info.num_lanes
NW = NC * NS
BPW = B // NW


def _sc_pool(x_flat, table):
    mesh = plsc.VectorSubcoreMesh(core_axis_name="c", subcore_axis_name="s")

    @functools.partial(
        pl.kernel,
        out_type=jax.ShapeDtypeStruct((B, D), jnp.float32),
        mesh=mesh,
        scratch_types=[
            pltpu.VMEM((BPW * L,), jnp.int32),
            pltpu.VMEM((L, D), jnp.float32),
            pltpu.VMEM((L, D), jnp.float32),
            pltpu.VMEM((BPW, D), jnp.float32),
            pltpu.SemaphoreType.DMA,
            pltpu.SemaphoreType.DMA,
        ],
        compiler_params=pltpu.CompilerParams(use_tc_tiling_on_sc=False),
    )
    def k(x_hbm, table_hbm, out_hbm, idx_v, rows0_v, rows1_v, pooled_v, sem0,
          sem1):
        wid = lax.axis_index("s") * NC + lax.axis_index("c")
        base = wid * (BPW * L)
        pltpu.sync_copy(x_hbm.at[pl.ds(base, BPW * L)], idx_v)

        def fire(j, buf, sem):
            off = pl.multiple_of(j * L, 8)
            pltpu.async_copy(
                table_hbm.at[idx_v.at[pl.ds(off, 96)]],
                buf.at[pl.ds(0, 96)], sem)
            pltpu.async_copy(
                table_hbm.at[idx_v.at[pl.ds(off + 96, 104)]],
                buf.at[pl.ds(96, 104)], sem)

        def drain(buf, sem):
            pltpu.make_async_copy(table_hbm.at[pl.ds(0, L)], buf, sem).wait()

        def accum(buf, j):
            def acc_body(i, carry):
                a0, a1, a2, a3 = carry
                l = i * 8
                for u in range(8):
                    a0 = a0 + buf[l + u, pl.ds(0, 16)]
                    a1 = a1 + buf[l + u, pl.ds(16, 16)]
                    a2 = a2 + buf[l + u, pl.ds(32, 16)]
                    a3 = a3 + buf[l + u, pl.ds(48, 16)]
                return a0, a1, a2, a3

            z = jnp.zeros((16,), jnp.float32)
            a0, a1, a2, a3 = lax.fori_loop(0, L // 8, acc_body, (z, z, z, z))
            scale = jnp.float32(1.0 / L)
            pooled_v[j, pl.ds(0, 16)] = a0 * scale
            pooled_v[j, pl.ds(16, 16)] = a1 * scale
            pooled_v[j, pl.ds(32, 16)] = a2 * scale
            pooled_v[j, pl.ds(48, 16)] = a3 * scale

        fire(0, rows0_v, sem0)
        fire(1, rows1_v, sem1)

        def pipe_body(i, _):
            j = i * 2
            drain(rows0_v, sem0)
            accum(rows0_v, j)
            fire(j + 2, rows0_v, sem0)
            drain(rows1_v, sem1)
            accum(rows1_v, j + 1)
            fire(j + 3, rows1_v, sem1)
            return 0

        lax.fori_loop(0, (BPW - 2) // 2, pipe_body, 0)
        drain(rows0_v, sem0)
        accum(rows0_v, BPW - 2)
        drain(rows1_v, sem1)
        accum(rows1_v, BPW - 1)
        pltpu.sync_copy(pooled_v, out_hbm.at[pl.ds(wid * BPW, BPW)])

    return k(x_flat, table)


def _mlp_body(p_ref, w1_ref, b1_ref, w2_ref, b2_ref, o_ref):
    h = jnp.dot(p_ref[...], w1_ref[...], preferred_element_type=jnp.float32)
    h = jnp.maximum(h + b1_ref[...], 0.0)
    o = jnp.dot(h, w2_ref[...], preferred_element_type=jnp.float32)
    o_ref[...] = jax.nn.sigmoid(o + b2_ref[...])


def _tc_mlp(pooled, W1, b1, W2, b2):
    return pl.pallas_call(
        _mlp_body,
        out_shape=jax.ShapeDtypeStruct((B, 1), jnp.float32),
    )(pooled, W1, b1.reshape(1, HID), W2, b2.reshape(1, 1))


def kernel(x, table, W1, b1, W2, b2):
    pooled = _sc_pool(x.reshape(-1), table)
    return _tc_mlp(pooled, W1, b1, W2, b2)


# speedup vs baseline: 16.6108x; 1.1755x over previous
import functools

import jax
import jax.numpy as jnp
from jax import lax
from jax.experimental import pallas as pl
from jax.experimental.pallas import tpu as pltpu
from jax.experimental.pallas import tpu_sc as plsc

VOCAB = 100000
D = 64
B = 4096
L = 200
HID = 100

_info = plsc.get_sparse_core_info()
NC, NS, LANES = _info.num_cores, _info.num_subcores, _info.num_lanes
NW = NC * NS
BPW = B // NW
NBUF = 3
UNROLL = 20


def _sc_pool(x, table):
    mesh = plsc.VectorSubcoreMesh(core_axis_name="c", subcore_axis_name="s")

    @functools.partial(
        pl.kernel,
        out_type=jax.ShapeDtypeStruct((B, D), jnp.float32),
        mesh=mesh,
        scratch_types=[
            pltpu.VMEM((BPW, L), jnp.int32),
            [pltpu.VMEM((L, D), jnp.float32) for _ in range(NBUF)],
            pltpu.VMEM((BPW, D), jnp.float32),
            [pltpu.SemaphoreType.DMA for _ in range(NBUF)],
        ],
        compiler_params=pltpu.CompilerParams(use_tc_tiling_on_sc=False),
    )
    def k(x_hbm, table_hbm, out_hbm, idx_v, bufs, pooled_v, sems):
        wid = lax.axis_index("s") * NC + lax.axis_index("c")
        pltpu.sync_copy(x_hbm.at[pl.ds(wid * BPW, BPW)], idx_v)

        def fire(j, buf, sem):
            pltpu.async_copy(
                table_hbm.at[idx_v.at[j, pl.ds(0, 96)]],
                buf.at[pl.ds(0, 96)], sem)
            pltpu.async_copy(
                table_hbm.at[idx_v.at[j, pl.ds(96, 104)]],
                buf.at[pl.ds(96, 104)], sem)

        def drain(buf, sem):
            pltpu.make_async_copy(table_hbm.at[pl.ds(0, L)], buf, sem).wait()

        def accum(buf, j):
            def acc_body(i, carry):
                a0, a1, a2, a3 = carry
                l = i * UNROLL
                for u in range(UNROLL):
                    a0 = a0 + buf[l + u, pl.ds(0, 16)]
                    a1 = a1 + buf[l + u, pl.ds(16, 16)]
                    a2 = a2 + buf[l + u, pl.ds(32, 16)]
                    a3 = a3 + buf[l + u, pl.ds(48, 16)]
                return a0, a1, a2, a3

            z = jnp.zeros((16,), jnp.float32)
            a0, a1, a2, a3 = lax.fori_loop(0, L // UNROLL, acc_body,
                                           (z, z, z, z))
            scale = jnp.float32(1.0 / L)
            pooled_v[j, pl.ds(0, 16)] = a0 * scale
            pooled_v[j, pl.ds(16, 16)] = a1 * scale
            pooled_v[j, pl.ds(32, 16)] = a2 * scale
            pooled_v[j, pl.ds(48, 16)] = a3 * scale

        for b in range(NBUF):
            fire(b, bufs[b], sems[b])

        def pipe_body(i, _):
            for b in range(NBUF):
                j = i * NBUF + b
                drain(bufs[b], sems[b])
                accum(bufs[b], j)

                @pl.when(j + NBUF < BPW)
                def _():
                    fire(j + NBUF, bufs[b], sems[b])
            return 0

        lax.fori_loop(0, BPW // NBUF, pipe_body, 0)
        for r in range(BPW - NBUF * (BPW // NBUF)):
            j = NBUF * (BPW // NBUF) + r
            drain(bufs[r], sems[r])
            accum(bufs[r], j)
        pltpu.sync_copy(pooled_v, out_hbm.at[pl.ds(wid * BPW, BPW)])

    return k(x, table)


def _mlp_body(p_ref, w1_ref, b1_ref, w2_ref, b2_ref, o_ref):
    h = jnp.dot(p_ref[...], w1_ref[...], preferred_element_type=jnp.float32)
    h = jnp.maximum(h + b1_ref[...], 0.0)
    o = jnp.dot(h, w2_ref[...], preferred_element_type=jnp.float32)
    o_ref[...] = jax.nn.sigmoid(o + b2_ref[...])


def _tc_mlp(pooled, W1, b1, W2, b2):
    return pl.pallas_call(
        _mlp_body,
        out_shape=jax.ShapeDtypeStruct((B, 1), jnp.float32),
    )(pooled, W1, b1.reshape(1, HID), W2, b2.reshape(1, 1))


def kernel(x, table, W1, b1, W2, b2):
    pooled = _sc_pool(x, table)
    return _tc_mlp(pooled, W1, b1, W2, b2)


# speedup vs baseline: 17.4195x; 1.0487x over previous
import functools

import jax
import jax.numpy as jnp
from jax import lax
from jax.experimental import pallas as pl
from jax.experimental.pallas import tpu as pltpu
from jax.experimental.pallas import tpu_sc as plsc

VOCAB = 100000
D = 64
B = 4096
L = 200
HID = 100

_info = plsc.get_sparse_core_info()
NC, NS, LANES = _info.num_cores, _info.num_subcores, _info.num_lanes
NW = NC * NS
BPW = B // NW
CHUNK = 128
NCHUNK = BPW * L // CHUNK
PAT = 25
RPP = PAT * CHUNK // L
NBUF = 5
UNROLL = 8


def _sc_pool(x2, table):
    mesh = plsc.VectorSubcoreMesh(core_axis_name="c", subcore_axis_name="s")

    @functools.partial(
        pl.kernel,
        out_type=jax.ShapeDtypeStruct((B, 2 * D), jnp.float32),
        mesh=mesh,
        scratch_types=[
            pltpu.VMEM((NCHUNK, CHUNK), jnp.int32),
            [pltpu.VMEM((CHUNK, D), jnp.float32) for _ in range(NBUF)],
            pltpu.VMEM((BPW, 2 * D), jnp.float32),
            [pltpu.SemaphoreType.DMA for _ in range(NBUF)],
        ],
        compiler_params=pltpu.CompilerParams(use_tc_tiling_on_sc=False),
    )
    def k(x_hbm, table_hbm, out_hbm, idx_v, bufs, pooled_v, sems):
        wid = lax.axis_index("s") * NC + lax.axis_index("c")
        pltpu.sync_copy(x_hbm.at[pl.ds(wid * NCHUNK, NCHUNK)], idx_v)

        def fire(crow, buf, sem):
            pltpu.async_copy(table_hbm.at[idx_v.at[crow]], buf, sem)

        def drain(buf, sem):
            pltpu.make_async_copy(table_hbm.at[pl.ds(0, CHUNK)], buf,
                                  sem).wait()

        def accum_range(buf, acc, start, end):
            def body(i, carry):
                a0, a1, a2, a3 = carry
                l = start + i * UNROLL
                for u in range(UNROLL):
                    a0 = a0 + buf[l + u, pl.ds(0, 16)]
                    a1 = a1 + buf[l + u, pl.ds(16, 16)]
                    a2 = a2 + buf[l + u, pl.ds(32, 16)]
                    a3 = a3 + buf[l + u, pl.ds(48, 16)]
                return a0, a1, a2, a3

            return lax.fori_loop(0, (end - start) // UNROLL, body, acc)

        z16 = jnp.zeros((16,), jnp.float32)
        zacc = (z16, z16, z16, z16)
        scale = jnp.float32(1.0 / L)

        def store_row(row, acc):
            a0, a1, a2, a3 = acc
            pooled_v[row, pl.ds(0, 16)] = a0 * scale
            pooled_v[row, pl.ds(16, 16)] = a1 * scale
            pooled_v[row, pl.ds(32, 16)] = a2 * scale
            pooled_v[row, pl.ds(48, 16)] = a3 * scale
            pooled_v[row, pl.ds(64, 16)] = z16
            pooled_v[row, pl.ds(80, 16)] = z16
            pooled_v[row, pl.ds(96, 16)] = z16
            pooled_v[row, pl.ds(112, 16)] = z16

        for b in range(NBUF):
            fire(b, bufs[b], sems[b])

        def super_body(i, _):
            acc = zacc
            m = 0
            for kk in range(PAT):
                b = kk % NBUF
                drain(bufs[b], sems[b])
                boundary = L * (m + 1) - CHUNK * kk
                if boundary < CHUNK:
                    if boundary > 0:
                        acc = accum_range(bufs[b], acc, 0, boundary)
                    store_row(i * RPP + m, acc)
                    m += 1
                    acc = accum_range(bufs[b], zacc, boundary, CHUNK)
                elif boundary == CHUNK:
                    acc = accum_range(bufs[b], acc, 0, CHUNK)
                    store_row(i * RPP + m, acc)
                    m += 1
                    acc = zacc
                else:
                    acc = accum_range(bufs[b], acc, 0, CHUNK)
                crow = i * PAT + kk

                @pl.when(crow + NBUF < NCHUNK)
                def _():
                    fire(crow + NBUF, bufs[b], sems[b])
            return 0

        lax.fori_loop(0, NCHUNK // PAT, super_body, 0)
        pltpu.sync_copy(pooled_v, out_hbm.at[pl.ds(wid * BPW, BPW)])

    return k(x2, table)


def _mlp_body(p_ref, w1_ref, b1_ref, w2_ref, b2_ref, o_ref):
    h = jnp.dot(p_ref[...], w1_ref[...], preferred_element_type=jnp.float32)
    h = jnp.maximum(h + b1_ref[...], 0.0)
    o = jnp.dot(h, w2_ref[...], preferred_element_type=jnp.float32)
    o_ref[...] = jax.nn.sigmoid(o + b2_ref[...])


def _tc_mlp(pooled, W1, b1, W2, b2):
    return pl.pallas_call(
        _mlp_body,
        out_shape=jax.ShapeDtypeStruct((B, 1), jnp.float32),
    )(pooled, W1, b1.reshape(1, HID), W2, b2.reshape(1, 1))


def kernel(x, table, W1, b1, W2, b2):
    pooled = _sc_pool(x.reshape(B * L // CHUNK, CHUNK), table)
    W1p = jnp.concatenate([W1, jnp.zeros((D, HID), jnp.float32)], axis=0)
    return _tc_mlp(pooled, W1p, b1, W2, b2)


# speedup vs baseline: 20.0420x; 1.1506x over previous
import functools

import jax
import jax.numpy as jnp
from jax import lax
from jax.experimental import pallas as pl
from jax.experimental.pallas import tpu as pltpu
from jax.experimental.pallas import tpu_sc as plsc

VOCAB = 100000
D = 64
B = 4096
L = 200
HID = 100

_info = plsc.get_sparse_core_info()
NC, NS, LANES = _info.num_cores, _info.num_subcores, _info.num_lanes
NW = NC * NS
BPW = B // NW
CHUNK = 128
NCHUNK = BPW * L // CHUNK
PAT = 25
RPP = PAT * CHUNK // L
NBUF = 5
UNROLL = 8


def _sc_pool(x2, table):
    mesh = plsc.VectorSubcoreMesh(core_axis_name="c", subcore_axis_name="s")

    @functools.partial(
        pl.kernel,
        out_type=jax.ShapeDtypeStruct((B, 2 * D), jnp.float32),
        mesh=mesh,
        scratch_types=[
            pltpu.VMEM((NCHUNK, CHUNK), jnp.int32),
            [pltpu.VMEM((CHUNK, D), jnp.float32) for _ in range(NBUF)],
            pltpu.VMEM((BPW, 2 * D), jnp.float32),
            [pltpu.SemaphoreType.DMA for _ in range(NBUF)],
        ],
        compiler_params=pltpu.CompilerParams(use_tc_tiling_on_sc=False),
    )
    def k(x_hbm, table_hbm, out_hbm, idx_v, bufs, pooled_v, sems):
        wid = lax.axis_index("s") * NC + lax.axis_index("c")
        pltpu.sync_copy(x_hbm.at[pl.ds(wid * NCHUNK, NCHUNK)], idx_v)

        def fire(crow, buf, sem):
            pltpu.async_copy(table_hbm.at[idx_v.at[crow]], buf, sem)

        def drain(buf, sem):
            pltpu.make_async_copy(table_hbm.at[pl.ds(0, CHUNK)], buf,
                                  sem).wait()

        def accum_range(buf, acc, start, end):
            def body(i, carry):
                a0, a1, a2, a3 = carry
                l = start + i * UNROLL
                for u in range(UNROLL):
                    a0 = a0 + buf[l + u, pl.ds(0, 16)]
                    a1 = a1 + buf[l + u, pl.ds(16, 16)]
                    a2 = a2 + buf[l + u, pl.ds(32, 16)]
                    a3 = a3 + buf[l + u, pl.ds(48, 16)]
                return a0, a1, a2, a3

            return lax.fori_loop(0, (end - start) // UNROLL, body, acc)

        z16 = jnp.zeros((16,), jnp.float32)
        zacc = (z16, z16, z16, z16)
        scale = jnp.float32(1.0 / L)

        def store_row(row, acc):
            a0, a1, a2, a3 = acc
            pooled_v[row, pl.ds(0, 16)] = a0 * scale
            pooled_v[row, pl.ds(16, 16)] = a1 * scale
            pooled_v[row, pl.ds(32, 16)] = a2 * scale
            pooled_v[row, pl.ds(48, 16)] = a3 * scale
            pooled_v[row, pl.ds(64, 16)] = z16
            pooled_v[row, pl.ds(80, 16)] = z16
            pooled_v[row, pl.ds(96, 16)] = z16
            pooled_v[row, pl.ds(112, 16)] = z16

        for b in range(NBUF):
            fire(b, bufs[b], sems[b])

        def super_body(i, _):
            acc = zacc
            m = 0
            for kk in range(PAT):
                b = kk % NBUF
                drain(bufs[b], sems[b])
                boundary = L * (m + 1) - CHUNK * kk
                if boundary < CHUNK:
                    if boundary > 0:
                        acc = accum_range(bufs[b], acc, 0, boundary)
                    store_row(i * RPP + m, acc)
                    m += 1
                    acc = accum_range(bufs[b], zacc, boundary, CHUNK)
                elif boundary == CHUNK:
                    acc = accum_range(bufs[b], acc, 0, CHUNK)
                    store_row(i * RPP + m, acc)
                    m += 1
                    acc = zacc
                else:
                    acc = accum_range(bufs[b], acc, 0, CHUNK)
                crow = i * PAT + kk

                @pl.when(crow + NBUF < NCHUNK)
                def _():
                    fire(crow + NBUF, bufs[b], sems[b])
            return 0

        lax.fori_loop(0, NCHUNK // PAT, super_body, 0)
        pltpu.sync_copy(pooled_v, out_hbm.at[pl.ds(wid * BPW, BPW)])

    return k(x2, table)


_DTW = 8192
_NDT = (VOCAB + _DTW - 1) // _DTW
VOCAB_P = _NDT * _DTW


def _detile_body(t_ref, o_ref):
    u = t_ref[...].T
    o_ref[...] = jnp.concatenate([u[: _DTW // 2], u[_DTW // 2 :]], axis=1)


def _tc_detile(table_t):
    return pl.pallas_call(
        _detile_body,
        grid=(_NDT,),
        in_specs=[pl.BlockSpec((D, _DTW), lambda i: (0, i))],
        out_specs=pl.BlockSpec((_DTW // 2, 128), lambda i: (i, 0)),
        out_shape=jax.ShapeDtypeStruct((VOCAB_P // 2, 128), jnp.float32),
    )(table_t)


def _mlp_body(p_ref, w1_ref, b1_ref, w2_ref, b2_ref, o_ref):
    h = jnp.dot(p_ref[...], w1_ref[...], preferred_element_type=jnp.float32)
    h = jnp.maximum(h + b1_ref[...], 0.0)
    o = jnp.dot(h, w2_ref[...], preferred_element_type=jnp.float32)
    o_ref[...] = jax.nn.sigmoid(o + b2_ref[...])


def _tc_mlp(pooled, W1, b1, W2, b2):
    return pl.pallas_call(
        _mlp_body,
        out_shape=jax.ShapeDtypeStruct((B, 1), jnp.float32),
    )(pooled, W1, b1.reshape(1, HID), W2, b2.reshape(1, 1))


def kernel(x, table, W1, b1, W2, b2):
    xm = (x & -_DTW) + ((x & (_DTW // 2 - 1)) << 1) + ((x >> 12) & 1)
    tbl = _tc_detile(jnp.transpose(table)).reshape(VOCAB_P, D)
    pooled = _sc_pool(xm.reshape(B * L // CHUNK, CHUNK), tbl)
    W1p = jnp.concatenate([W1, jnp.zeros((D, HID), jnp.float32)], axis=0)
    return _tc_mlp(pooled, W1p, b1, W2, b2)
